# baseline (speedup 1.0000x reference)
import functools

import jax
import jax.numpy as jnp
from jax.experimental import pallas as pl
from jax.experimental.pallas import tpu as pltpu

_NUM_TYPES = 10
_E = 32
_K = 16
_OVER = 5.0
_TL = 512

_MU = (0.10000000149011612, 0.20000000298023224, 0.30000001192092896,
       0.4000000059604645, 0.5, 0.6000000238418579, 0.699999988079071,
       0.800000011920929, 0.8999999761581421, 1.0)


def _rotl(x, r):
    return (x << jnp.uint32(r)) | (x >> jnp.uint32(32 - r))


def _threefry_bits(k1_int, x1):
    k1 = jnp.uint32(k1_int)
    ks2 = jnp.uint32(0x1BD11BDA) ^ k1
    zero = jnp.uint32(0)
    x0 = jnp.zeros_like(x1)
    x1 = x1 + k1
    rots0 = (13, 15, 26, 6)
    rots1 = (17, 29, 16, 24)
    inj = ((k1, ks2), (ks2, zero), (zero, k1), (k1, ks2), (ks2, zero))
    for g in range(5):
        for r in (rots0 if g % 2 == 0 else rots1):
            x0 = x0 + x1
            x1 = _rotl(x1, r)
            x1 = x1 ^ x0
        a, bb = inj[g]
        x0 = x0 + a
        x1 = x1 + (bb + jnp.uint32(g + 1))
    return x0 ^ x1


def _bits_to_uniform(bits):
    f = jax.lax.bitcast_convert_type(
        (bits >> jnp.uint32(9)) | jnp.uint32(0x3F800000), jnp.float32)
    return f - jnp.float32(1.0)


def _body(t_ref, dt_ref, ty_ref, out_ref, *, L):
    b = pl.program_id(0)
    lt = pl.program_id(1)
    t = t_ref[0]
    dt = dt_ref[0]
    ty = ty_ref[0]

    te = jnp.zeros_like(t)
    for k in range(_NUM_TYPES):
        te = te + jnp.where(ty == k, jnp.float32(_MU[k]), jnp.float32(0.0))

    base = jnp.float32(0.1) + jax.nn.softplus(
        te + jnp.float32(0.1) * dt + jnp.float32(0.01) * jnp.cos(t))

    v0 = jnp.zeros_like(base)
    for k in range(_NUM_TYPES):
        v0 = v0 + (base * jnp.float32(_MU[k]) + jnp.float32(0.05))
    M = v0 * jnp.float32(_OVER)

    sub = jax.lax.broadcasted_iota(jnp.int32, (_E, _TL), 0)
    lane = jax.lax.broadcasted_iota(jnp.int32, (_E, _TL), 1)
    l0 = lt * _TL
    ie = (b * (L * _E) + (l0 + lane) * _E + sub).astype(jnp.uint32)
    u1 = _bits_to_uniform(_threefry_bits(1, ie))
    e = -jnp.log1p(-u1)
    x = e / M
    for s in (1, 2, 4, 8, 16):
        shifted = jnp.concatenate(
            [jnp.zeros((s, _TL), jnp.float32), x[:-s, :]], axis=0)
        x = x + shifted
    exp_j = x

    st = base * jnp.exp(jnp.float32(-0.5) * exp_j)
    intens = jnp.zeros_like(st)
    for k in range(_NUM_TYPES):
        intens = intens + (st * jnp.float32(_MU[k]) + jnp.float32(0.05))

    rows = []
    big = jnp.float32(jnp.inf)
    for k in range(_K):
        iu = (b * (L * _K * _E) + (l0 + lane) * (_K * _E)
              + k * _E + sub).astype(jnp.uint32)
        u = _bits_to_uniform(_threefry_bits(2, iu))
        crit = (u * M) / intens
        cand = jnp.where(crit < jnp.float32(1.0), exp_j, big)
        mval = jnp.min(cand, axis=0, keepdims=True)
        res = jnp.where(mval == big, jnp.float32(0.0),
                        jnp.minimum(mval, jnp.float32(100000.0)))
        rows.append(res)
    kt = jnp.concatenate(rows, axis=0)
    out_ref[...] = jnp.transpose(kt)[None]


def kernel(time_seqs, time_delta_seqs, type_seqs, num_sample):
    B, L = time_seqs.shape
    in_spec = pl.BlockSpec((1, 1, _TL), lambda b, lt: (b, 0, lt))
    res = pl.pallas_call(
        functools.partial(_body, L=L),
        grid=(B, L // _TL),
        in_specs=[in_spec, in_spec, in_spec],
        out_specs=pl.BlockSpec((1, _TL, _K), lambda b, lt: (b, lt, 0)),
        out_shape=jax.ShapeDtypeStruct((B, L, _K), jnp.float32),
        compiler_params=pltpu.CompilerParams(
            dimension_semantics=("parallel", "parallel")),
    )(time_seqs.reshape(B, 1, L), time_delta_seqs.reshape(B, 1, L),
      type_seqs.reshape(B, 1, L))
    weights = jnp.ones((B, L, _K), jnp.float32) / num_sample
    return (res, weights)


# speedup vs baseline: 1.0793x; 1.0793x over previous
import functools

import jax
import jax.numpy as jnp
from jax.experimental import pallas as pl
from jax.experimental.pallas import tpu as pltpu

_NUM_TYPES = 10
_E = 32
_K = 16
_OVER = 5.0
_TL = 512

_MU = (0.10000000149011612, 0.20000000298023224, 0.30000001192092896,
       0.4000000059604645, 0.5, 0.6000000238418579, 0.699999988079071,
       0.800000011920929, 0.8999999761581421, 1.0)


def _rotl(x, r):
    return (x << jnp.uint32(r)) | (x >> jnp.uint32(32 - r))


def _threefry_bits(k1_int, x1):
    k1i = k1_int & 0xFFFFFFFF
    ks2i = (0x1BD11BDA ^ k1i) & 0xFFFFFFFF
    x0 = jnp.zeros_like(x1)
    x1 = x1 + jnp.uint32(k1i)
    rots0 = (13, 15, 26, 6)
    rots1 = (17, 29, 16, 24)
    inj = ((k1i, ks2i + 1), (ks2i, 2), (0, k1i + 3), (k1i, ks2i + 4),
           (ks2i, 5))
    for g in range(5):
        for r in (rots0 if g % 2 == 0 else rots1):
            x0 = x0 + x1
            x1 = _rotl(x1, r)
            x1 = x1 ^ x0
        a, bb = inj[g]
        if a:
            x0 = x0 + jnp.uint32(a)
        x1 = x1 + jnp.uint32(bb & 0xFFFFFFFF)
    return x0 ^ x1


def _bits_to_uniform(bits):
    f = jax.lax.bitcast_convert_type(
        (bits >> jnp.uint32(9)) | jnp.uint32(0x3F800000), jnp.float32)
    return f - jnp.float32(1.0)


def _body(t_ref, dt_ref, ty_ref, out_ref, *, L):
    b = pl.program_id(0)
    lt = pl.program_id(1)
    t = t_ref[0]
    dt = dt_ref[0]
    ty = ty_ref[0]

    te = jnp.zeros_like(t)
    for k in range(_NUM_TYPES):
        te = te + jnp.where(ty == k, jnp.float32(_MU[k]), jnp.float32(0.0))

    base = jnp.float32(0.1) + jax.nn.softplus(
        te + jnp.float32(0.1) * dt + jnp.float32(0.01) * jnp.cos(t))

    v0 = jnp.zeros_like(base)
    for k in range(_NUM_TYPES):
        v0 = v0 + (base * jnp.float32(_MU[k]) + jnp.float32(0.05))
    M = v0 * jnp.float32(_OVER)

    sub = jax.lax.broadcasted_iota(jnp.int32, (_E, _TL), 0)
    lane = jax.lax.broadcasted_iota(jnp.int32, (_E, _TL), 1)
    l0 = lt * _TL
    ie = (b * (L * _E) + (l0 + lane) * _E + sub).astype(jnp.uint32)
    u1 = _bits_to_uniform(_threefry_bits(1, ie))
    e = -jnp.log1p(-u1)
    x = e / M
    for s in (1, 2, 4, 8, 16):
        shifted = jnp.concatenate(
            [jnp.zeros((s, _TL), jnp.float32), x[:-s, :]], axis=0)
        x = x + shifted
    exp_j = x

    st = base * jnp.exp(jnp.float32(-0.5) * exp_j)
    intens = jnp.zeros_like(st)
    for k in range(_NUM_TYPES):
        intens = intens + (st * jnp.float32(_MU[k]) + jnp.float32(0.05))

    thr = (intens / M) * jnp.float32(8388608.0)

    rows = []
    big = jnp.float32(jnp.inf)
    iu0 = b * (L * _K * _E) + (l0 + lane) * (_K * _E) + sub
    for k in range(_K):
        iu = (iu0 + k * _E).astype(jnp.uint32)
        mant = _threefry_bits(2, iu) >> jnp.uint32(9)
        mf = mant.astype(jnp.int32).astype(jnp.float32)
        cand = jnp.where(mf < thr, exp_j, big)
        mval = jnp.min(cand, axis=0, keepdims=True)
        res = jnp.where(mval == big, jnp.float32(0.0),
                        jnp.minimum(mval, jnp.float32(100000.0)))
        rows.append(res)
    out_ref[...] = jnp.concatenate(rows, axis=0)


def kernel(time_seqs, time_delta_seqs, type_seqs, num_sample):
    B, L = time_seqs.shape
    in_spec = pl.BlockSpec((1, 1, _TL), lambda b, lt: (b, 0, lt))
    out = pl.pallas_call(
        functools.partial(_body, L=L),
        grid=(B, L // _TL),
        in_specs=[in_spec, in_spec, in_spec],
        out_specs=pl.BlockSpec((_K, _TL), lambda b, lt: (b, lt)),
        out_shape=jax.ShapeDtypeStruct((B * _K, L), jnp.float32),
        compiler_params=pltpu.CompilerParams(
            dimension_semantics=("parallel", "parallel")),
    )(time_seqs.reshape(B, 1, L), time_delta_seqs.reshape(B, 1, L),
      type_seqs.reshape(B, 1, L))
    res = out.reshape(B, _K, L).transpose(0, 2, 1)
    weights = jnp.ones((B, L, _K), jnp.float32) / num_sample
    return (res, weights)


# speedup vs baseline: 1.0888x; 1.0087x over previous
import functools

import jax
import jax.numpy as jnp
from jax.experimental import pallas as pl
from jax.experimental.pallas import tpu as pltpu

_NUM_TYPES = 10
_E = 32
_K = 16
_OVER = 5.0
_TL = 1024

_MU = (0.10000000149011612, 0.20000000298023224, 0.30000001192092896,
       0.4000000059604645, 0.5, 0.6000000238418579, 0.699999988079071,
       0.800000011920929, 0.8999999761581421, 1.0)


def _rotl(x, r):
    return (x << jnp.uint32(r)) | (x >> jnp.uint32(32 - r))


def _threefry_bits(k1_int, x1):
    k1i = k1_int & 0xFFFFFFFF
    ks2i = (0x1BD11BDA ^ k1i) & 0xFFFFFFFF
    x0 = jnp.zeros_like(x1)
    x1 = x1 + jnp.uint32(k1i)
    rots0 = (13, 15, 26, 6)
    rots1 = (17, 29, 16, 24)
    inj = ((k1i, ks2i + 1), (ks2i, 2), (0, k1i + 3), (k1i, ks2i + 4),
           (ks2i, 5))
    for g in range(5):
        for r in (rots0 if g % 2 == 0 else rots1):
            x0 = x0 + x1
            x1 = _rotl(x1, r)
            x1 = x1 ^ x0
        a, bb = inj[g]
        if a:
            x0 = x0 + jnp.uint32(a)
        x1 = x1 + jnp.uint32(bb & 0xFFFFFFFF)
    return x0 ^ x1


def _bits_to_uniform(bits):
    f = jax.lax.bitcast_convert_type(
        (bits >> jnp.uint32(9)) | jnp.uint32(0x3F800000), jnp.float32)
    return f - jnp.float32(1.0)


def _body(t_ref, dt_ref, ty_ref, out_ref, *, L):
    b = pl.program_id(0)
    lt = pl.program_id(1)
    t = t_ref[0]
    dt = dt_ref[0]
    ty = ty_ref[0]

    te = jnp.zeros_like(t)
    for k in range(_NUM_TYPES):
        te = te + jnp.where(ty == k, jnp.float32(_MU[k]), jnp.float32(0.0))

    base = jnp.float32(0.1) + jax.nn.softplus(
        te + jnp.float32(0.1) * dt + jnp.float32(0.01) * jnp.cos(t))

    v0 = jnp.zeros_like(base)
    for k in range(_NUM_TYPES):
        v0 = v0 + (base * jnp.float32(_MU[k]) + jnp.float32(0.05))
    M = v0 * jnp.float32(_OVER)

    sub = jax.lax.broadcasted_iota(jnp.int32, (_E, _TL), 0)
    lane = jax.lax.broadcasted_iota(jnp.int32, (_E, _TL), 1)
    l0 = lt * _TL
    ie = (b * (L * _E) + (l0 + lane) * _E + sub).astype(jnp.uint32)
    u1 = _bits_to_uniform(_threefry_bits(1, ie))
    e = -jnp.log1p(-u1)
    x = e / M
    for s in (1, 2, 4, 8, 16):
        shifted = jnp.concatenate(
            [jnp.zeros((s, _TL), jnp.float32), x[:-s, :]], axis=0)
        x = x + shifted
    exp_j = x

    st = base * jnp.exp(jnp.float32(-0.5) * exp_j)
    intens = jnp.zeros_like(st)
    for k in range(_NUM_TYPES):
        intens = intens + (st * jnp.float32(_MU[k]) + jnp.float32(0.05))

    thr = (intens / M) * jnp.float32(8388608.0)

    rows = []
    big = jnp.float32(jnp.inf)
    iu0 = b * (L * _K * _E) + (l0 + lane) * (_K * _E) + sub
    for k in range(_K):
        iu = (iu0 + k * _E).astype(jnp.uint32)
        mant = _threefry_bits(2, iu) >> jnp.uint32(9)
        mf = mant.astype(jnp.int32).astype(jnp.float32)
        cand = jnp.where(mf < thr, exp_j, big)
        mval = jnp.min(cand, axis=0, keepdims=True)
        res = jnp.where(mval == big, jnp.float32(0.0),
                        jnp.minimum(mval, jnp.float32(100000.0)))
        rows.append(res)
    out_ref[...] = jnp.concatenate(rows, axis=0)


def kernel(time_seqs, time_delta_seqs, type_seqs, num_sample):
    B, L = time_seqs.shape
    in_spec = pl.BlockSpec((1, 1, _TL), lambda b, lt: (b, 0, lt))
    out = pl.pallas_call(
        functools.partial(_body, L=L),
        grid=(B, L // _TL),
        in_specs=[in_spec, in_spec, in_spec],
        out_specs=pl.BlockSpec((_K, _TL), lambda b, lt: (b, lt)),
        out_shape=jax.ShapeDtypeStruct((B * _K, L), jnp.float32),
        compiler_params=pltpu.CompilerParams(
            dimension_semantics=("parallel", "parallel")),
    )(time_seqs.reshape(B, 1, L), time_delta_seqs.reshape(B, 1, L),
      type_seqs.reshape(B, 1, L))
    res = out.reshape(B, _K, L).transpose(0, 2, 1)
    weights = jnp.ones((B, L, _K), jnp.float32) / num_sample
    return (res, weights)


# speedup vs baseline: 1.0905x; 1.0016x over previous
import functools

import jax
import jax.numpy as jnp
from jax.experimental import pallas as pl
from jax.experimental.pallas import tpu as pltpu

_NUM_TYPES = 10
_E = 32
_K = 16
_OVER = 5.0
_TL = 2048

_MU = (0.10000000149011612, 0.20000000298023224, 0.30000001192092896,
       0.4000000059604645, 0.5, 0.6000000238418579, 0.699999988079071,
       0.800000011920929, 0.8999999761581421, 1.0)


def _rotl(x, r):
    return (x << jnp.uint32(r)) | (x >> jnp.uint32(32 - r))


def _threefry_bits(k1_int, x1):
    k1i = k1_int & 0xFFFFFFFF
    ks2i = (0x1BD11BDA ^ k1i) & 0xFFFFFFFF
    x0 = jnp.zeros_like(x1)
    x1 = x1 + jnp.uint32(k1i)
    rots0 = (13, 15, 26, 6)
    rots1 = (17, 29, 16, 24)
    inj = ((k1i, ks2i + 1), (ks2i, 2), (0, k1i + 3), (k1i, ks2i + 4),
           (ks2i, 5))
    for g in range(5):
        for r in (rots0 if g % 2 == 0 else rots1):
            x0 = x0 + x1
            x1 = _rotl(x1, r)
            x1 = x1 ^ x0
        a, bb = inj[g]
        if a:
            x0 = x0 + jnp.uint32(a)
        x1 = x1 + jnp.uint32(bb & 0xFFFFFFFF)
    return x0 ^ x1


def _bits_to_uniform(bits):
    f = jax.lax.bitcast_convert_type(
        (bits >> jnp.uint32(9)) | jnp.uint32(0x3F800000), jnp.float32)
    return f - jnp.float32(1.0)


def _body(t_ref, dt_ref, ty_ref, out_ref, *, L):
    b = pl.program_id(0)
    lt = pl.program_id(1)
    t = t_ref[0]
    dt = dt_ref[0]
    ty = ty_ref[0]

    te = jnp.zeros_like(t)
    for k in range(_NUM_TYPES):
        te = te + jnp.where(ty == k, jnp.float32(_MU[k]), jnp.float32(0.0))

    base = jnp.float32(0.1) + jax.nn.softplus(
        te + jnp.float32(0.1) * dt + jnp.float32(0.01) * jnp.cos(t))

    v0 = jnp.zeros_like(base)
    for k in range(_NUM_TYPES):
        v0 = v0 + (base * jnp.float32(_MU[k]) + jnp.float32(0.05))
    M = v0 * jnp.float32(_OVER)

    sub = jax.lax.broadcasted_iota(jnp.int32, (_E, _TL), 0)
    lane = jax.lax.broadcasted_iota(jnp.int32, (_E, _TL), 1)
    l0 = lt * _TL
    ie = (b * (L * _E) + (l0 + lane) * _E + sub).astype(jnp.uint32)
    u1 = _bits_to_uniform(_threefry_bits(1, ie))
    e = -jnp.log1p(-u1)
    x = e / M
    for s in (1, 2, 4, 8, 16):
        shifted = jnp.concatenate(
            [jnp.zeros((s, _TL), jnp.float32), x[:-s, :]], axis=0)
        x = x + shifted
    exp_j = x

    st = base * jnp.exp(jnp.float32(-0.5) * exp_j)
    intens = jnp.zeros_like(st)
    for k in range(_NUM_TYPES):
        intens = intens + (st * jnp.float32(_MU[k]) + jnp.float32(0.05))

    thr = (intens / M) * jnp.float32(8388608.0)

    rows = []
    big = jnp.float32(jnp.inf)
    iu0 = b * (L * _K * _E) + (l0 + lane) * (_K * _E) + sub
    for k in range(_K):
        iu = (iu0 + k * _E).astype(jnp.uint32)
        mant = _threefry_bits(2, iu) >> jnp.uint32(9)
        mf = mant.astype(jnp.int32).astype(jnp.float32)
        cand = jnp.where(mf < thr, exp_j, big)
        mval = jnp.min(cand, axis=0, keepdims=True)
        res = jnp.where(mval == big, jnp.float32(0.0),
                        jnp.minimum(mval, jnp.float32(100000.0)))
        rows.append(res)
    out_ref[...] = jnp.concatenate(rows, axis=0)


def kernel(time_seqs, time_delta_seqs, type_seqs, num_sample):
    B, L = time_seqs.shape
    in_spec = pl.BlockSpec((1, 1, _TL), lambda b, lt: (b, 0, lt))
    out = pl.pallas_call(
        functools.partial(_body, L=L),
        grid=(B, L // _TL),
        in_specs=[in_spec, in_spec, in_spec],
        out_specs=pl.BlockSpec((_K, _TL), lambda b, lt: (b, lt)),
        out_shape=jax.ShapeDtypeStruct((B * _K, L), jnp.float32),
        compiler_params=pltpu.CompilerParams(
            dimension_semantics=("parallel", "parallel")),
    )(time_seqs.reshape(B, 1, L), time_delta_seqs.reshape(B, 1, L),
      type_seqs.reshape(B, 1, L))
    res = out.reshape(B, _K, L).transpose(0, 2, 1)
    weights = jnp.ones((B, L, _K), jnp.float32) / num_sample
    return (res, weights)


# speedup vs baseline: 1.2690x; 1.1637x over previous
import functools

import jax
import jax.numpy as jnp
from jax.experimental import pallas as pl
from jax.experimental.pallas import tpu as pltpu
from jax._src.pallas.mosaic import sc_core as plsc
from jax._src.pallas.mosaic import sc_primitives as plscp

_NUM_TYPES = 10
_E = 32
_K = 16
_OVER = 5.0
_TL = 2048
_B_SC = 4
_N_SUBCORES = 32

_MU = (0.10000000149011612, 0.20000000298023224, 0.30000001192092896,
       0.4000000059604645, 0.5, 0.6000000238418579, 0.699999988079071,
       0.800000011920929, 0.8999999761581421, 1.0)


def _rotl(x, r):
    return (x << jnp.uint32(r)) | (x >> jnp.uint32(32 - r))


def _threefry_bits(k1_int, x1):
    k1i = k1_int & 0xFFFFFFFF
    ks2i = (0x1BD11BDA ^ k1i) & 0xFFFFFFFF
    x0 = jnp.zeros_like(x1)
    x1 = x1 + jnp.uint32(k1i)
    rots0 = (13, 15, 26, 6)
    rots1 = (17, 29, 16, 24)
    inj = ((k1i, ks2i + 1), (ks2i, 2), (0, k1i + 3), (k1i, ks2i + 4),
           (ks2i, 5))
    for g in range(5):
        for r in (rots0 if g % 2 == 0 else rots1):
            x0 = x0 + x1
            x1 = _rotl(x1, r)
            x1 = x1 ^ x0
        a, bb = inj[g]
        if a:
            x0 = x0 + jnp.uint32(a)
        x1 = x1 + jnp.uint32(bb & 0xFFFFFFFF)
    return x0 ^ x1


def _bits_to_uniform(bits):
    f = jax.lax.bitcast_convert_type(
        (bits >> jnp.uint32(9)) | jnp.uint32(0x3F800000), jnp.float32)
    return f - jnp.float32(1.0)


def _row_stats(t, dt, ty):
    te = jnp.zeros_like(t)
    for k in range(_NUM_TYPES):
        te = te + jnp.where(ty == k, jnp.float32(_MU[k]), jnp.float32(0.0))
    base = jnp.float32(0.1) + jax.nn.softplus(
        te + jnp.float32(0.1) * dt + jnp.float32(0.01) * jnp.cos(t))
    v0 = jnp.zeros_like(base)
    for k in range(_NUM_TYPES):
        v0 = v0 + (base * jnp.float32(_MU[k]) + jnp.float32(0.05))
    M = v0 * jnp.float32(_OVER)
    return base, M


def _expj_thr(b, l0, t, dt, ty, W, L):
    base, M = _row_stats(t, dt, ty)
    sub = jax.lax.broadcasted_iota(jnp.int32, (_E, W), 0)
    lane = jax.lax.broadcasted_iota(jnp.int32, (_E, W), 1)
    ie = (b * (L * _E) + (l0 + lane) * _E + sub).astype(jnp.uint32)
    u1 = _bits_to_uniform(_threefry_bits(1, ie))
    e = -jnp.log1p(-u1)
    x = e / M
    for s in (1, 2, 4, 8, 16):
        shifted = jnp.concatenate(
            [jnp.zeros((s, W), jnp.float32), x[:-s, :]], axis=0)
        x = x + shifted
    exp_j = x
    st = base * jnp.exp(jnp.float32(-0.5) * exp_j)
    intens = jnp.zeros_like(st)
    for k in range(_NUM_TYPES):
        intens = intens + (st * jnp.float32(_MU[k]) + jnp.float32(0.05))
    thr = (intens / M) * jnp.float32(8388608.0)
    return exp_j, thr, lane, sub


def _tc_body(t_ref, dt_ref, ty_ref, out_ref, *, L):
    b = pl.program_id(0)
    lt = pl.program_id(1)
    l0 = lt * _TL
    exp_j, thr, lane, sub = _expj_thr(
        b, l0, t_ref[0], dt_ref[0], ty_ref[0], _TL, L)
    rows = []
    big = jnp.float32(jnp.inf)
    iu0 = b * (L * _K * _E) + (l0 + lane) * (_K * _E) + sub
    for k in range(_K):
        iu = (iu0 + k * _E).astype(jnp.uint32)
        mant = _threefry_bits(2, iu) >> jnp.uint32(9)
        mf = mant.astype(jnp.int32).astype(jnp.float32)
        cand = jnp.where(mf < thr, exp_j, big)
        mval = jnp.min(cand, axis=0, keepdims=True)
        res = jnp.where(mval == big, jnp.float32(0.0),
                        jnp.minimum(mval, jnp.float32(100000.0)))
        rows.append(res)
    out_ref[...] = jnp.concatenate(rows, axis=0)


def _prep_body(t_ref, dt_ref, ty_ref, ej_ref, it_ref, *, L, B_TC):
    b = pl.program_id(0) + B_TC
    exp_j, thr, _, _ = _expj_thr(
        b, 0, t_ref[0], dt_ref[0], ty_ref[0], L, L)
    ej_ref[...] = exp_j
    it_ref[...] = jnp.ceil(thr).astype(jnp.int32)


def _sc_body(ej_hbm, it_hbm, out_hbm, ev, tv, ov, *, L, B_TC, CH):
    wid = jax.lax.axis_index("c") * 16 + jax.lax.axis_index("s")
    chunks_per_row = L // CH
    b = wid // chunks_per_row
    c0 = (wid % chunks_per_row) * CH
    pltpu.sync_copy(ej_hbm.at[pl.ds(b * _E, _E), pl.ds(c0, CH)], ev)
    pltpu.sync_copy(it_hbm.at[pl.ds(b * _E, _E), pl.ds(c0, CH)], tv)
    lanes = jax.lax.iota(jnp.int32, 16)
    big = jnp.full((16,), jnp.inf, jnp.float32)
    zero = jnp.zeros((16,), jnp.float32)
    cap = jnp.full((16,), 100000.0, jnp.float32)
    row_base = (b + B_TC) * (L * _K * _E)

    def step(i, carry):
        lg = i // _K
        k = i % _K
        col = c0 + lg * 16
        cbase = row_base + (col + lanes) * (_K * _E) + k * _E
        macc = big
        for j in range(_E):
            bits = _threefry_bits(2, (cbase + j).astype(jnp.uint32))
            mant = bits >> jnp.uint32(9)
            tvec = tv[j, pl.ds(lg * 16, 16)]
            evec = ev[j, pl.ds(lg * 16, 16)]
            ok = mant < plscp.bitcast(tvec, jnp.uint32)
            macc = jnp.minimum(macc, jnp.where(ok, evec, big))
        res = jnp.where(macc == big, zero, jnp.minimum(macc, cap))
        ov[k, pl.ds(lg * 16, 16)] = res
        return carry

    jax.lax.fori_loop(0, (CH // 16) * _K, step, 0)
    pltpu.sync_copy(ov, out_hbm.at[pl.ds(b * _K, _K), pl.ds(c0, CH)])


def kernel(time_seqs, time_delta_seqs, type_seqs, num_sample):
    B, L = time_seqs.shape
    B_TC = B - _B_SC
    CH = L * _B_SC // _N_SUBCORES
    t3 = time_seqs.reshape(B, 1, L)
    dt3 = time_delta_seqs.reshape(B, 1, L)
    ty3 = type_seqs.reshape(B, 1, L)

    prep_in = pl.BlockSpec((1, 1, L), lambda i: (i + B_TC, 0, 0))
    ej, it = pl.pallas_call(
        functools.partial(_prep_body, L=L, B_TC=B_TC),
        grid=(_B_SC,),
        in_specs=[prep_in, prep_in, prep_in],
        out_specs=[pl.BlockSpec((_E, L), lambda i: (i, 0)),
                   pl.BlockSpec((_E, L), lambda i: (i, 0))],
        out_shape=[jax.ShapeDtypeStruct((_B_SC * _E, L), jnp.float32),
                   jax.ShapeDtypeStruct((_B_SC * _E, L), jnp.int32)],
    )(t3, dt3, ty3)

    sc_fn = pl.kernel(
        functools.partial(_sc_body, L=L, B_TC=B_TC, CH=CH),
        out_type=jax.ShapeDtypeStruct((_B_SC * _K, L), jnp.float32),
        mesh=plsc.VectorSubcoreMesh(core_axis_name="c",
                                    subcore_axis_name="s"),
        scratch_types=[pltpu.VMEM((_E, CH), jnp.float32),
                       pltpu.VMEM((_E, CH), jnp.int32),
                       pltpu.VMEM((_K, CH), jnp.float32)],
    )
    out_sc = sc_fn(ej, it)

    in_spec = pl.BlockSpec((1, 1, _TL), lambda b, lt: (b, 0, lt))
    out_tc = pl.pallas_call(
        functools.partial(_tc_body, L=L),
        grid=(B_TC, L // _TL),
        in_specs=[in_spec, in_spec, in_spec],
        out_specs=pl.BlockSpec((_K, _TL), lambda b, lt: (b, lt)),
        out_shape=jax.ShapeDtypeStruct((B_TC * _K, L), jnp.float32),
        compiler_params=pltpu.CompilerParams(
            dimension_semantics=("parallel", "parallel")),
    )(t3[:B_TC], dt3[:B_TC], ty3[:B_TC])

    res = jnp.concatenate(
        [out_tc.reshape(B_TC, _K, L).transpose(0, 2, 1),
         out_sc.reshape(_B_SC, _K, L).transpose(0, 2, 1)], axis=0)
    weights = jnp.ones((B, L, _K), jnp.float32) / num_sample
    return (res, weights)
